# Initial kernel scaffold; baseline (speedup 1.0000x reference)
import jax
import jax.numpy as jnp
from jax.experimental import pallas as pl


def kernel(input, hidden1, W_i2m, b_i2m, W_m2h, b_m2h, W_h2o, b_h2o):
    raise NotImplementedError("write your pallas kernel here")


# speedup vs baseline: 67.4836x; 67.4836x over previous
import functools

import jax
import jax.numpy as jnp
from jax import lax
from jax.experimental import pallas as pl
from jax.experimental.pallas import tpu as pltpu
from jax.experimental.pallas import tpu_sc as plsc

TB = 256
ROWS_B = 512
CHUNK = 64


def _argmax_body(x_ref, w1t_ref, b1_ref, w2t_ref, b2_ref, idx_ref):
    h = jnp.maximum(
        jnp.dot(x_ref[...], w1t_ref[...], preferred_element_type=jnp.float32)
        + b1_ref[...], 0.0)
    s = jnp.dot(h, w2t_ref[...], preferred_element_type=jnp.float32) + b2_ref[...]
    m = jnp.max(s, axis=-1, keepdims=True)
    col = lax.broadcasted_iota(jnp.int32, s.shape, 1)
    idx = jnp.min(jnp.where(s == m, col, jnp.int32(2**30)), axis=-1)
    idx_ref[0, 0, :] = idx


def _logsoftmax_body(wt_ref, b_ref, out_ref):
    z = wt_ref[...] + b_ref[...]
    m = jnp.max(z, axis=-1, keepdims=True)
    e = jnp.exp(z - m)
    lse = m + jnp.log(jnp.sum(e, axis=-1, keepdims=True))
    out_ref[...] = z - lse


def _make_gather(n_tokens, d):
    info = plsc.get_sparse_core_info()
    nc, ns = info.num_cores, info.num_subcores
    nw = nc * ns
    b_per_w = n_tokens // nw
    n_chunks = b_per_w // CHUNK
    mesh = plsc.VectorSubcoreMesh(core_axis_name="c", subcore_axis_name="s")

    @functools.partial(
        pl.kernel,
        mesh=mesh,
        out_type=jax.ShapeDtypeStruct((n_tokens, d), jnp.float32),
        scratch_types=[
            pltpu.VMEM((CHUNK,), jnp.int32),
            pltpu.VMEM((CHUNK, d), jnp.float32),
            pltpu.SemaphoreType.DMA,
        ],
    )
    def gather_k(table_hbm, idx_hbm, out_hbm, idx_v, rows_v, sem):
        wid = lax.axis_index("s") * nc + lax.axis_index("c")
        base = wid * b_per_w

        def body(i, carry):
            off = base + i * CHUNK
            pltpu.sync_copy(idx_hbm.at[pl.ds(off, CHUNK)], idx_v)
            pltpu.async_copy(table_hbm.at[idx_v], rows_v, sem).wait()
            pltpu.sync_copy(rows_v, out_hbm.at[pl.ds(off, CHUNK)])
            return carry

        lax.fori_loop(0, n_chunks, body, 0)

    return gather_k


def kernel(input, hidden1, W_i2m, b_i2m, W_m2h, b_m2h, W_h2o, b_h2o):
    B, S, I = input.shape
    N = B * S
    H = W_i2m.shape[0]
    C = W_m2h.shape[0]
    O = W_h2o.shape[0]

    x = input.reshape(N, I)
    w1t = W_i2m.T
    w2t = W_m2h.T
    b1 = b_i2m.reshape(1, H)
    b2 = b_m2h.reshape(1, C)

    nb = N // TB
    idx3 = pl.pallas_call(
        _argmax_body,
        grid=(nb,),
        in_specs=[
            pl.BlockSpec((TB, I), lambda i: (i, 0)),
            pl.BlockSpec((I, H), lambda i: (0, 0)),
            pl.BlockSpec((1, H), lambda i: (0, 0)),
            pl.BlockSpec((H, C), lambda i: (0, 0)),
            pl.BlockSpec((1, C), lambda i: (0, 0)),
        ],
        out_specs=pl.BlockSpec((1, 1, TB), lambda i: (i, 0, 0)),
        out_shape=jax.ShapeDtypeStruct((nb, 1, TB), jnp.int32),
    )(x, w1t, b1, w2t, b2)
    idx = idx3.reshape(N)

    wt = W_h2o.T
    bo = b_h2o.reshape(1, O)
    table = pl.pallas_call(
        _logsoftmax_body,
        grid=(C // ROWS_B,),
        in_specs=[
            pl.BlockSpec((ROWS_B, O), lambda i: (i, 0)),
            pl.BlockSpec((1, O), lambda i: (0, 0)),
        ],
        out_specs=pl.BlockSpec((ROWS_B, O), lambda i: (i, 0)),
        out_shape=jax.ShapeDtypeStruct((C, O), jnp.float32),
    )(wt, bo)

    out = _make_gather(N, O)(table, idx)
    return out.reshape(B, S, O)
